# baseline (speedup 1.0000x reference)
import functools

import jax
import jax.numpy as jnp
from jax import lax
from jax.experimental import pallas as pl
from jax.experimental.pallas import tpu as pltpu
from jax.experimental.pallas import tpu_sc as plsc

_DIM = 768
_HID = 2048
_E = 8
_SEQ = 2048
_NPAIR = 2 * _SEQ
_BLK = 256
_N = _NPAIR + _E * _BLK
_NB = _N // _BLK
_NTILE = 32
_JCHUNK = _NPAIR // _NTILE
_TCHUNK = _SEQ // _NTILE


def _router_body(x_ref, gw_ref, pos_ref, wpair_ref, bexp_ref, bval_ref):
    x = x_ref[...]
    logits = lax.dot_general(x, gw_ref[...], (((1,), (1,)), ((), ())),
                             preferred_element_type=jnp.float32)
    m = jnp.max(logits, axis=1, keepdims=True)
    p = jnp.exp(logits - m)
    p = p / jnp.sum(p, axis=1, keepdims=True)

    lane = lax.broadcasted_iota(jnp.int32, p.shape, 1)
    m1 = jnp.max(p, axis=1, keepdims=True)
    i1 = jnp.min(jnp.where(p == m1, lane, _E), axis=1, keepdims=True)
    p_wo = jnp.where(lane == i1, -1.0, p)
    m2 = jnp.max(p_wo, axis=1, keepdims=True)
    i2 = jnp.min(jnp.where(p_wo == m2, lane, _E), axis=1, keepdims=True)
    mask1 = (lane == i1).astype(jnp.float32)
    mask2 = (lane == i2).astype(jnp.float32)
    wfull = (mask1 + mask2) * p / (m1 + m2)

    sub = lax.broadcasted_iota(jnp.int32, (_E, _E), 0)
    eye = (sub == lax.broadcasted_iota(jnp.int32, (_E, _E), 1)).astype(
        jnp.float32)
    tr = lambda a: lax.dot_general(eye, a, (((1,), (1,)), ((), ())),
                                   preferred_element_type=jnp.float32)
    c1 = tr(mask1)
    c2 = tr(mask2)
    wT = tr(wfull)
    c = jnp.concatenate([c1, c2], axis=1)
    wpair = jnp.sum(jnp.concatenate([c1 * wT, c2 * wT], axis=1), axis=0,
                    keepdims=True)

    incl = c
    d = 1
    while d < _NPAIR:
        z = jnp.zeros((_E, d), jnp.float32)
        incl = incl + jnp.concatenate([z, incl[:, : _NPAIR - d]], axis=1)
        d *= 2
    rankx = incl - c
    counts = incl[:, _NPAIR - 1:_NPAIR]
    padded = jnp.floor((counts + (_BLK - 1)) * (1.0 / _BLK)) * _BLK
    poff = padded
    d = 1
    while d < _E:
        z = jnp.zeros((d, 1), jnp.float32)
        poff = poff + jnp.concatenate([z, poff[: _E - d, :]], axis=0)
        d *= 2
    off = poff - padded
    total = jnp.sum(padded, axis=0, keepdims=True)

    pos = jnp.sum(c * (off + rankx), axis=0, keepdims=True)
    pos_ref[...] = pos.astype(jnp.int32)
    wpair_ref[...] = wpair

    bb = (lax.broadcasted_iota(jnp.int32, (1, _NB), 1) * _BLK).astype(
        jnp.float32)
    subi = lax.broadcasted_iota(jnp.int32, (_E, _NB), 0)
    hit = ((off <= bb) & (subi >= 1)).astype(jnp.float32)
    bexp_ref[...] = jnp.sum(hit, axis=0, keepdims=True).astype(jnp.int32)
    bval_ref[...] = (bb < total).astype(jnp.int32)


def _router(x, gate_w):
    return pl.pallas_call(
        _router_body,
        in_specs=[
            pl.BlockSpec((_SEQ, _DIM), lambda: (0, 0)),
            pl.BlockSpec((_E, _DIM), lambda: (0, 0)),
        ],
        out_specs=[
            pl.BlockSpec((1, _NPAIR), lambda: (0, 0)),
            pl.BlockSpec((1, _NPAIR), lambda: (0, 0)),
            pl.BlockSpec((1, _NB), lambda: (0, 0)),
            pl.BlockSpec((1, _NB), lambda: (0, 0)),
        ],
        out_shape=[
            jax.ShapeDtypeStruct((1, _NPAIR), jnp.int32),
            jax.ShapeDtypeStruct((1, _NPAIR), jnp.float32),
            jax.ShapeDtypeStruct((1, _NB), jnp.int32),
            jax.ShapeDtypeStruct((1, _NB), jnp.int32),
        ],
    )(x, gate_w)


_HCH = _JCHUNK // 2


_WCH = _NPAIR // 16
_SSL = _N // _NTILE


def _dispatch_body(x_hbm, pos_hbm, wpair_hbm, xs_hbm, wsort_hbm,
                   idx_v, rows0_v, rows1_v, idxw_v, wpw_v, wsl_v, wsort_sp,
                   s_pos, s_w, s_r0, s_r1, s_x0, s_x1):
    c = lax.axis_index("c")
    s = lax.axis_index("s")
    wid = s * 2 + c
    base = wid * _JCHUNK
    tbase = lax.rem(base, _SEQ)
    wbase = s * _WCH

    cp_r0 = pltpu.async_copy(x_hbm.at[pl.ds(tbase, _HCH), :], rows0_v, s_r0)
    cp_r1 = pltpu.async_copy(x_hbm.at[pl.ds(tbase + _HCH, _HCH), :],
                             rows1_v, s_r1)
    cp_p0 = pltpu.async_copy(pos_hbm.at[pl.ds(base, _HCH)], idx_v.at[0],
                             s_pos)
    cp_p1 = pltpu.async_copy(pos_hbm.at[pl.ds(base + _HCH, _HCH)],
                             idx_v.at[1], s_pos)
    cp_q0 = pltpu.async_copy(pos_hbm.at[pl.ds(wbase, 128)], idxw_v.at[0],
                             s_w)
    cp_q1 = pltpu.async_copy(pos_hbm.at[pl.ds(wbase + 128, 128)],
                             idxw_v.at[1], s_w)
    cp_w0 = pltpu.async_copy(wpair_hbm.at[pl.ds(wbase, 128)], wpw_v.at[0],
                             s_w)
    cp_w1 = pltpu.async_copy(wpair_hbm.at[pl.ds(wbase + 128, 128)],
                             wpw_v.at[1], s_w)
    cp_p0.wait()
    cp_p1.wait()
    cp_r0.wait()
    sc_x0 = pltpu.async_copy(rows0_v, xs_hbm.at[idx_v.at[0]], s_x0)
    cp_r1.wait()
    sc_x1 = pltpu.async_copy(rows1_v, xs_hbm.at[idx_v.at[1]], s_x1)
    cp_q0.wait()
    cp_q1.wait()
    cp_w0.wait()
    cp_w1.wait()
    pltpu.sync_copy(wpw_v.at[0], wsort_sp.at[idxw_v.at[0]])
    pltpu.sync_copy(wpw_v.at[1], wsort_sp.at[idxw_v.at[1]])
    plsc.subcore_barrier()
    pltpu.sync_copy(wsort_sp.at[pl.ds(wid * _SSL, _SSL)], wsl_v)
    pltpu.sync_copy(wsl_v, wsort_hbm.at[pl.ds(wid * _SSL, _SSL)])
    sc_x0.wait()
    sc_x1.wait()


def _dispatch(x, pos, wpair):
    mesh = plsc.VectorSubcoreMesh(core_axis_name="c", subcore_axis_name="s")
    f = pl.kernel(
        _dispatch_body,
        out_type=[
            jax.ShapeDtypeStruct((_N, _DIM), jnp.float32),
            jax.ShapeDtypeStruct((_N,), jnp.float32),
        ],
        mesh=mesh,
        scratch_types=[
            pltpu.VMEM((2, _HCH), jnp.int32),
            pltpu.VMEM((_HCH, _DIM), jnp.float32),
            pltpu.VMEM((_HCH, _DIM), jnp.float32),
            pltpu.VMEM((2, 128), jnp.int32),
            pltpu.VMEM((2, 128), jnp.float32),
            pltpu.VMEM((_SSL,), jnp.float32),
            pltpu.VMEM_SHARED((_N,), jnp.float32),
            pltpu.SemaphoreType.DMA,
            pltpu.SemaphoreType.DMA,
            pltpu.SemaphoreType.DMA,
            pltpu.SemaphoreType.DMA,
            pltpu.SemaphoreType.DMA,
            pltpu.SemaphoreType.DMA,
        ],
    )
    return f(x, pos, wpair)


def _gemm_body(bexp_ref, bval_ref, xs_ref, ws_ref, w1_ref, w2_ref, w3_ref,
               yw_ref):
    b = pl.program_id(0)

    @pl.when(bval_ref[b] == 1)
    def _():
        xsb = xs_ref[...]
        h1 = lax.dot_general(xsb, w1_ref[0], (((1,), (1,)), ((), ())),
                             preferred_element_type=jnp.float32)
        h3 = lax.dot_general(xsb, w3_ref[0], (((1,), (1,)), ((), ())),
                             preferred_element_type=jnp.float32)
        hh = (h1 * (1.0 / (1.0 + jnp.exp(-h1)))) * h3
        y = lax.dot_general(hh, w2_ref[0], (((1,), (1,)), ((), ())),
                            preferred_element_type=jnp.float32)
        yw_ref[...] = y * ws_ref[...]


def _gemm(bexp, bval, xs, wsort2d, w1, w2, w3):
    grid_spec = pltpu.PrefetchScalarGridSpec(
        num_scalar_prefetch=2,
        grid=(_NB,),
        in_specs=[
            pl.BlockSpec((_BLK, _DIM), lambda b, be, bv: (b, 0)),
            pl.BlockSpec((_BLK, 1), lambda b, be, bv: (b, 0)),
            pl.BlockSpec((1, _HID, _DIM), lambda b, be, bv: (be[b], 0, 0)),
            pl.BlockSpec((1, _DIM, _HID), lambda b, be, bv: (be[b], 0, 0)),
            pl.BlockSpec((1, _HID, _DIM), lambda b, be, bv: (be[b], 0, 0)),
        ],
        out_specs=pl.BlockSpec((_BLK, _DIM), lambda b, be, bv: (b, 0)),
    )
    return pl.pallas_call(
        _gemm_body,
        grid_spec=grid_spec,
        out_shape=jax.ShapeDtypeStruct((_N, _DIM), jnp.float32),
    )(bexp, bval, xs, wsort2d, w1, w2, w3)


def _combine_body(yw_hbm, pos_hbm, out_hbm, pa_v, pb_v, za_v, zb_v,
                  sem1, sem2):
    c = lax.axis_index("c")
    s = lax.axis_index("s")
    wid = s * 2 + c
    tb = wid * _TCHUNK
    pltpu.sync_copy(pos_hbm.at[pl.ds(tb, _TCHUNK)], pa_v)
    pltpu.sync_copy(pos_hbm.at[pl.ds(_SEQ + tb, _TCHUNK)], pb_v)
    cp1 = pltpu.async_copy(yw_hbm.at[pa_v], za_v, sem1)
    cp2 = pltpu.async_copy(yw_hbm.at[pb_v], zb_v, sem2)
    cp1.wait()
    cp2.wait()

    def row_add(i, carry):
        for d in range(_DIM // 16):
            sl = pl.ds(d * 16, 16)
            za_v[i, sl] = za_v[i, sl] + zb_v[i, sl]
        return carry

    lax.fori_loop(0, _TCHUNK, row_add, 0)
    pltpu.sync_copy(za_v, out_hbm.at[pl.ds(tb, _TCHUNK), :])


def _combine(yw, pos):
    mesh = plsc.VectorSubcoreMesh(core_axis_name="c", subcore_axis_name="s")
    f = pl.kernel(
        _combine_body,
        out_type=jax.ShapeDtypeStruct((_SEQ, _DIM), jnp.float32),
        mesh=mesh,
        scratch_types=[
            pltpu.VMEM((_TCHUNK,), jnp.int32),
            pltpu.VMEM((_TCHUNK,), jnp.int32),
            pltpu.VMEM((_TCHUNK, _DIM), jnp.float32),
            pltpu.VMEM((_TCHUNK, _DIM), jnp.float32),
            pltpu.SemaphoreType.DMA,
            pltpu.SemaphoreType.DMA,
        ],
    )
    return f(yw, pos)


def kernel(hidden_states, gate_w, w1, w2, w3):
    bs, seq, dim = hidden_states.shape
    x = hidden_states.reshape(seq, dim)
    pos2d, wpair2d, bexp2d, bval2d = _router(x, gate_w)
    pos = pos2d.reshape(_NPAIR)
    wpair = wpair2d.reshape(_NPAIR)
    bexp = bexp2d.reshape(_NB)
    bval = bval2d.reshape(_NB)
    xs, wsort = _dispatch(x, pos, wpair)
    yw = _gemm(bexp, bval, xs, wsort.reshape(_N, 1), w1, w2, w3)
    out = _combine(yw, pos)
    return out.reshape(bs, seq, dim)


# speedup vs baseline: 1.1545x; 1.1545x over previous
import functools

import jax
import jax.numpy as jnp
from jax import lax
from jax.experimental import pallas as pl
from jax.experimental.pallas import tpu as pltpu
from jax.experimental.pallas import tpu_sc as plsc

_DIM = 768
_HID = 2048
_E = 8
_SEQ = 2048
_NPAIR = 2 * _SEQ
_BLK = 256
_N = _NPAIR + _E * _BLK
_NB = _N // _BLK
_NTILE = 32
_JCHUNK = _NPAIR // _NTILE
_TCHUNK = _SEQ // _NTILE


def _router_body(x_ref, gw_ref, pos_ref, wpair_ref, bval_ref, rstart_ref,
                 ridx_ref, rexp_ref):
    x = x_ref[...]
    logits = lax.dot_general(x, gw_ref[...], (((1,), (1,)), ((), ())),
                             preferred_element_type=jnp.float32)
    m = jnp.max(logits, axis=1, keepdims=True)
    p = jnp.exp(logits - m)
    p = p / jnp.sum(p, axis=1, keepdims=True)

    lane = lax.broadcasted_iota(jnp.int32, p.shape, 1)
    m1 = jnp.max(p, axis=1, keepdims=True)
    i1 = jnp.min(jnp.where(p == m1, lane, _E), axis=1, keepdims=True)
    p_wo = jnp.where(lane == i1, -1.0, p)
    m2 = jnp.max(p_wo, axis=1, keepdims=True)
    i2 = jnp.min(jnp.where(p_wo == m2, lane, _E), axis=1, keepdims=True)
    mask1 = (lane == i1).astype(jnp.float32)
    mask2 = (lane == i2).astype(jnp.float32)
    wfull = (mask1 + mask2) * p / (m1 + m2)

    sub = lax.broadcasted_iota(jnp.int32, (_E, _E), 0)
    eye = (sub == lax.broadcasted_iota(jnp.int32, (_E, _E), 1)).astype(
        jnp.float32)
    tr = lambda a: lax.dot_general(eye, a, (((1,), (1,)), ((), ())),
                                   preferred_element_type=jnp.float32)
    c1 = tr(mask1)
    c2 = tr(mask2)
    c = jnp.concatenate([c1, c2], axis=1)
    wT = tr(wfull)
    wpair = jnp.sum(jnp.concatenate([c1 * wT, c2 * wT], axis=1), axis=0,
                    keepdims=True)

    incl = c
    d = 1
    while d < _NPAIR:
        z = jnp.zeros((_E, d), jnp.float32)
        incl = incl + jnp.concatenate([z, incl[:, : _NPAIR - d]], axis=1)
        d *= 2
    rankx = incl - c
    counts = incl[:, _NPAIR - 1:_NPAIR]
    padded = jnp.floor((counts + (_BLK - 1)) * (1.0 / _BLK)) * _BLK
    poff = padded
    d = 1
    while d < _E:
        z = jnp.zeros((d, 1), jnp.float32)
        poff = poff + jnp.concatenate([z, poff[: _E - d, :]], axis=0)
        d *= 2
    off = poff - padded
    total = jnp.sum(padded, axis=0, keepdims=True)

    pos = jnp.sum(c * (off + rankx), axis=0, keepdims=True)
    pos_ref[...] = pos.astype(jnp.int32)[0]
    wpair_ref[...] = wpair[0]

    bb = (lax.broadcasted_iota(jnp.int32, (1, _NB), 1) * _BLK).astype(
        jnp.float32)
    subi = lax.broadcasted_iota(jnp.int32, (_E, _NB), 0)
    hit = ((off <= bb) & (subi >= 1)).astype(jnp.float32)
    bexp = jnp.sum(hit, axis=0, keepdims=True)
    bval = (bb < total).astype(jnp.float32)
    bval_ref[...] = bval.astype(jnp.int32)[0]

    lane_nb = lax.broadcasted_iota(jnp.int32, (1, _NB), 1)
    bexp_prev = jnp.concatenate(
        [jnp.full((1, 1), -1.0, jnp.float32), bexp[:, : _NB - 1]], axis=1)
    rstart = bval * jnp.where((bexp != bexp_prev) | (lane_nb == 0), 1.0,
                              0.0)
    ridx = rstart
    d = 1
    while d < _NB:
        z = jnp.zeros((1, d), jnp.float32)
        ridx = ridx + jnp.concatenate([z, ridx[:, : _NB - d]], axis=1)
        d *= 2
    ridx = ridx - 1.0
    rstart_ref[...] = rstart.astype(jnp.int32)[0]
    ridx_ref[...] = jnp.maximum(ridx, 0.0).astype(jnp.int32)[0]

    nz = (padded > 0).astype(jnp.float32)
    rr = nz
    d = 1
    while d < _E:
        z = jnp.zeros((d, 1), jnp.float32)
        rr = rr + jnp.concatenate([z, rr[: _E - d, :]], axis=0)
        d *= 2
    r_e = rr - 1.0
    ii = lax.broadcasted_iota(jnp.int32, (_E, 16), 1).astype(jnp.float32)
    match = jnp.where((r_e == ii) & (padded > 0), 1.0, 0.0)
    eidx = lax.broadcasted_iota(jnp.int32, (_E, 16), 0).astype(jnp.float32)
    raw = jnp.sum(eidx * match, axis=0, keepdims=True)
    has = jnp.sum(match, axis=0, keepdims=True)
    rexp_ref[...] = jnp.where(has > 0, raw, -1.0).astype(jnp.int32)[0]


def _router(x, gate_w):
    return pl.pallas_call(
        _router_body,
        in_specs=[
            pl.BlockSpec((_SEQ, _DIM), lambda: (0, 0)),
            pl.BlockSpec((_E, _DIM), lambda: (0, 0)),
        ],
        out_specs=[
            pl.BlockSpec((_NPAIR,), lambda: (0,)),
            pl.BlockSpec((_NPAIR,), lambda: (0,)),
            pl.BlockSpec((_NB,), lambda: (0,)),
            pl.BlockSpec((_NB,), lambda: (0,)),
            pl.BlockSpec((_NB,), lambda: (0,)),
            pl.BlockSpec((16,), lambda: (0,)),
        ],
        out_shape=[
            jax.ShapeDtypeStruct((_NPAIR,), jnp.int32),
            jax.ShapeDtypeStruct((_NPAIR,), jnp.float32),
            jax.ShapeDtypeStruct((_NB,), jnp.int32),
            jax.ShapeDtypeStruct((_NB,), jnp.int32),
            jax.ShapeDtypeStruct((_NB,), jnp.int32),
            jax.ShapeDtypeStruct((16,), jnp.int32),
        ],
    )(x, gate_w)


_HCH = _JCHUNK // 2


_WCH = _NPAIR // 16
_SSL = _N // _NTILE


def _dispatch_body(x_hbm, pos_hbm, wpair_hbm, xs_hbm, wsort_hbm,
                   idx_v, rows0_v, rows1_v, idxw_v, wpw_v, wsl_v, wsort_sp,
                   s_pos, s_w, s_r0, s_r1, s_x0, s_x1):
    c = lax.axis_index("c")
    s = lax.axis_index("s")
    wid = s * 2 + c
    base = wid * _JCHUNK
    tbase = lax.rem(base, _SEQ)
    wbase = s * _WCH

    cp_r0 = pltpu.async_copy(x_hbm.at[pl.ds(tbase, _HCH), :], rows0_v, s_r0)
    cp_r1 = pltpu.async_copy(x_hbm.at[pl.ds(tbase + _HCH, _HCH), :],
                             rows1_v, s_r1)
    cp_p0 = pltpu.async_copy(pos_hbm.at[pl.ds(base, _HCH)], idx_v.at[0],
                             s_pos)
    cp_p1 = pltpu.async_copy(pos_hbm.at[pl.ds(base + _HCH, _HCH)],
                             idx_v.at[1], s_pos)
    cp_q0 = pltpu.async_copy(pos_hbm.at[pl.ds(wbase, 128)], idxw_v.at[0],
                             s_w)
    cp_q1 = pltpu.async_copy(pos_hbm.at[pl.ds(wbase + 128, 128)],
                             idxw_v.at[1], s_w)
    cp_w0 = pltpu.async_copy(wpair_hbm.at[pl.ds(wbase, 128)], wpw_v.at[0],
                             s_w)
    cp_w1 = pltpu.async_copy(wpair_hbm.at[pl.ds(wbase + 128, 128)],
                             wpw_v.at[1], s_w)
    cp_p0.wait()
    cp_p1.wait()
    cp_r0.wait()
    sc_x0 = pltpu.async_copy(rows0_v, xs_hbm.at[idx_v.at[0]], s_x0)
    cp_r1.wait()
    sc_x1 = pltpu.async_copy(rows1_v, xs_hbm.at[idx_v.at[1]], s_x1)
    cp_q0.wait()
    cp_q1.wait()
    cp_w0.wait()
    cp_w1.wait()
    pltpu.sync_copy(wpw_v.at[0], wsort_sp.at[idxw_v.at[0]])
    pltpu.sync_copy(wpw_v.at[1], wsort_sp.at[idxw_v.at[1]])
    plsc.subcore_barrier()
    pltpu.sync_copy(wsort_sp.at[pl.ds(wid * _SSL, _SSL)], wsl_v)
    pltpu.sync_copy(wsl_v, wsort_hbm.at[pl.ds(wid * _SSL, _SSL)])
    sc_x0.wait()
    sc_x1.wait()


def _dispatch(x, pos, wpair):
    mesh = plsc.VectorSubcoreMesh(core_axis_name="c", subcore_axis_name="s")
    f = pl.kernel(
        _dispatch_body,
        out_type=[
            jax.ShapeDtypeStruct((_N, _DIM), jnp.float32),
            jax.ShapeDtypeStruct((_N,), jnp.float32),
        ],
        mesh=mesh,
        scratch_types=[
            pltpu.VMEM((2, _HCH), jnp.int32),
            pltpu.VMEM((_HCH, _DIM), jnp.float32),
            pltpu.VMEM((_HCH, _DIM), jnp.float32),
            pltpu.VMEM((2, 128), jnp.int32),
            pltpu.VMEM((2, 128), jnp.float32),
            pltpu.VMEM((_SSL,), jnp.float32),
            pltpu.VMEM_SHARED((_N,), jnp.float32),
            pltpu.SemaphoreType.DMA,
            pltpu.SemaphoreType.DMA,
            pltpu.SemaphoreType.DMA,
            pltpu.SemaphoreType.DMA,
            pltpu.SemaphoreType.DMA,
            pltpu.SemaphoreType.DMA,
        ],
    )
    return f(x, pos, wpair)


def _gemm_body(bval_ref, rstart_ref, ridx_ref, rexp_ref,
               xs_ref, ws_ref, w1_hbm, w2_hbm, w3_hbm, yw_ref,
               w1b, w2b, w3b, sems):
    b = pl.program_id(0)
    i = ridx_ref[b]
    slot = lax.rem(i, 2)

    def _issue(run_i, slot_i):
        e = rexp_ref[run_i]

        @pl.when(e >= 0)
        def _():
            pltpu.make_async_copy(w1_hbm.at[e], w1b.at[slot_i],
                                  sems.at[slot_i, 0]).start()
            pltpu.make_async_copy(w2_hbm.at[e], w2b.at[slot_i],
                                  sems.at[slot_i, 1]).start()
            pltpu.make_async_copy(w3_hbm.at[e], w3b.at[slot_i],
                                  sems.at[slot_i, 2]).start()

    def _wait(run_i, slot_i):
        e = jnp.maximum(rexp_ref[run_i], 0)
        pltpu.make_async_copy(w1_hbm.at[e], w1b.at[slot_i],
                              sems.at[slot_i, 0]).wait()
        pltpu.make_async_copy(w2_hbm.at[e], w2b.at[slot_i],
                              sems.at[slot_i, 1]).wait()
        pltpu.make_async_copy(w3_hbm.at[e], w3b.at[slot_i],
                              sems.at[slot_i, 2]).wait()

    @pl.when(b == 0)
    def _():
        _issue(0, 0)
        _issue(1, 1)
        _wait(0, 0)

    @pl.when((rstart_ref[b] == 1) & (b > 0))
    def _():
        _wait(i, slot)
        _issue(i + 1, lax.rem(i + 1, 2))

    @pl.when(bval_ref[b] == 1)
    def _():
        xsb = xs_ref[...]
        h1 = lax.dot_general(xsb, w1b[slot], (((1,), (1,)), ((), ())),
                             preferred_element_type=jnp.float32)
        h3 = lax.dot_general(xsb, w3b[slot], (((1,), (1,)), ((), ())),
                             preferred_element_type=jnp.float32)
        hh = (h1 * (1.0 / (1.0 + jnp.exp(-h1)))) * h3
        y = lax.dot_general(hh, w2b[slot], (((1,), (1,)), ((), ())),
                            preferred_element_type=jnp.float32)
        yw_ref[...] = y * ws_ref[...]


def _gemm(bval, rstart, ridx, rexp, xs, wsort, w1, w2, w3):
    grid_spec = pltpu.PrefetchScalarGridSpec(
        num_scalar_prefetch=4,
        grid=(_NB,),
        in_specs=[
            pl.BlockSpec((_BLK, _DIM), lambda b, *_: (b, 0)),
            pl.BlockSpec((_BLK, 1), lambda b, *_: (b, 0)),
            pl.BlockSpec(memory_space=pl.ANY),
            pl.BlockSpec(memory_space=pl.ANY),
            pl.BlockSpec(memory_space=pl.ANY),
        ],
        out_specs=pl.BlockSpec((_BLK, _DIM), lambda b, *_: (b, 0)),
        scratch_shapes=[
            pltpu.VMEM((2, _HID, _DIM), jnp.float32),
            pltpu.VMEM((2, _DIM, _HID), jnp.float32),
            pltpu.VMEM((2, _HID, _DIM), jnp.float32),
            pltpu.SemaphoreType.DMA((2, 3)),
        ],
    )
    return pl.pallas_call(
        _gemm_body,
        grid_spec=grid_spec,
        out_shape=jax.ShapeDtypeStruct((_N, _DIM), jnp.float32),
    )(bval, rstart, ridx, rexp, xs, wsort, w1, w2, w3)


def _combine_body(yw_hbm, pos_hbm, out_hbm, pa_v, pb_v, za_v, zb_v,
                  sem1, sem2):
    c = lax.axis_index("c")
    s = lax.axis_index("s")
    wid = s * 2 + c
    tb = wid * _TCHUNK
    pltpu.sync_copy(pos_hbm.at[pl.ds(tb, _TCHUNK)], pa_v)
    pltpu.sync_copy(pos_hbm.at[pl.ds(_SEQ + tb, _TCHUNK)], pb_v)
    cp1 = pltpu.async_copy(yw_hbm.at[pa_v], za_v, sem1)
    cp2 = pltpu.async_copy(yw_hbm.at[pb_v], zb_v, sem2)
    cp1.wait()
    cp2.wait()

    def row_add(i, carry):
        for d in range(_DIM // 16):
            sl = pl.ds(d * 16, 16)
            za_v[i, sl] = za_v[i, sl] + zb_v[i, sl]
        return carry

    lax.fori_loop(0, _TCHUNK, row_add, 0)
    pltpu.sync_copy(za_v, out_hbm.at[pl.ds(tb, _TCHUNK), :])


def _combine(yw, pos):
    mesh = plsc.VectorSubcoreMesh(core_axis_name="c", subcore_axis_name="s")
    f = pl.kernel(
        _combine_body,
        out_type=jax.ShapeDtypeStruct((_SEQ, _DIM), jnp.float32),
        mesh=mesh,
        scratch_types=[
            pltpu.VMEM((_TCHUNK,), jnp.int32),
            pltpu.VMEM((_TCHUNK,), jnp.int32),
            pltpu.VMEM((_TCHUNK, _DIM), jnp.float32),
            pltpu.VMEM((_TCHUNK, _DIM), jnp.float32),
            pltpu.SemaphoreType.DMA,
            pltpu.SemaphoreType.DMA,
        ],
    )
    return f(yw, pos)


def kernel(hidden_states, gate_w, w1, w2, w3):
    bs, seq, dim = hidden_states.shape
    x = hidden_states.reshape(seq, dim)
    pos, wpair, bval, rstart, ridx, rexp = _router(x, gate_w)
    xs, wsort = _dispatch(x, pos, wpair)
    yw = _gemm(bval, rstart, ridx, rexp, xs, wsort.reshape(_N, 1), w1, w2,
               w3)
    out = _combine(yw, pos)
    return out.reshape(bs, seq, dim)


# speedup vs baseline: 1.1653x; 1.0094x over previous
import functools

import jax
import jax.numpy as jnp
from jax import lax
from jax.experimental import pallas as pl
from jax.experimental.pallas import tpu as pltpu
from jax.experimental.pallas import tpu_sc as plsc

_DIM = 768
_HID = 2048
_E = 8
_SEQ = 2048
_NPAIR = 2 * _SEQ
_BLK = 256
_N = _NPAIR + _E * _BLK
_NB = _N // _BLK
_NTILE = 32
_JCHUNK = _NPAIR // _NTILE
_TCHUNK = _SEQ // _NTILE


def _router_body(x_ref, gw_ref, pos_ref, wpair_ref, bval_ref, rstart_ref,
                 ridx_ref, rexp_ref):
    x = x_ref[...]
    logits = lax.dot_general(x, gw_ref[...], (((1,), (1,)), ((), ())),
                             preferred_element_type=jnp.float32)
    m = jnp.max(logits, axis=1, keepdims=True)
    p = jnp.exp(logits - m)
    p = p / jnp.sum(p, axis=1, keepdims=True)

    lane = lax.broadcasted_iota(jnp.int32, p.shape, 1)
    m1 = jnp.max(p, axis=1, keepdims=True)
    i1 = jnp.min(jnp.where(p == m1, lane, _E), axis=1, keepdims=True)
    p_wo = jnp.where(lane == i1, -1.0, p)
    m2 = jnp.max(p_wo, axis=1, keepdims=True)
    i2 = jnp.min(jnp.where(p_wo == m2, lane, _E), axis=1, keepdims=True)
    mask1 = (lane == i1).astype(jnp.float32)
    mask2 = (lane == i2).astype(jnp.float32)
    wfull = (mask1 + mask2) * p / (m1 + m2)

    sub = lax.broadcasted_iota(jnp.int32, (_E, _E), 0)
    eye = (sub == lax.broadcasted_iota(jnp.int32, (_E, _E), 1)).astype(
        jnp.float32)
    tr = lambda a: lax.dot_general(eye, a, (((1,), (1,)), ((), ())),
                                   preferred_element_type=jnp.float32)
    c1 = tr(mask1)
    c2 = tr(mask2)
    c = jnp.concatenate([c1, c2], axis=1)
    wT = tr(wfull)
    wpair = jnp.sum(jnp.concatenate([c1 * wT, c2 * wT], axis=1), axis=0,
                    keepdims=True)

    incl = c
    d = 1
    while d < _NPAIR:
        z = jnp.zeros((_E, d), jnp.float32)
        incl = incl + jnp.concatenate([z, incl[:, : _NPAIR - d]], axis=1)
        d *= 2
    rankx = incl - c
    counts = incl[:, _NPAIR - 1:_NPAIR]
    padded = jnp.floor((counts + (_BLK - 1)) * (1.0 / _BLK)) * _BLK
    poff = padded
    d = 1
    while d < _E:
        z = jnp.zeros((d, 1), jnp.float32)
        poff = poff + jnp.concatenate([z, poff[: _E - d, :]], axis=0)
        d *= 2
    off = poff - padded
    total = jnp.sum(padded, axis=0, keepdims=True)

    pos = jnp.sum(c * (off + rankx), axis=0, keepdims=True)
    pos_ref[...] = pos.astype(jnp.int32)[0]
    wpair_ref[...] = wpair[0]

    bb = (lax.broadcasted_iota(jnp.int32, (1, _NB), 1) * _BLK).astype(
        jnp.float32)
    subi = lax.broadcasted_iota(jnp.int32, (_E, _NB), 0)
    hit = ((off <= bb) & (subi >= 1)).astype(jnp.float32)
    bexp = jnp.sum(hit, axis=0, keepdims=True)
    bval = (bb < total).astype(jnp.float32)
    bval_ref[...] = bval.astype(jnp.int32)[0]

    lane_nb = lax.broadcasted_iota(jnp.int32, (1, _NB), 1)
    bexp_prev = jnp.concatenate(
        [jnp.full((1, 1), -1.0, jnp.float32), bexp[:, : _NB - 1]], axis=1)
    rstart = bval * jnp.where((bexp != bexp_prev) | (lane_nb == 0), 1.0,
                              0.0)
    ridx = rstart
    d = 1
    while d < _NB:
        z = jnp.zeros((1, d), jnp.float32)
        ridx = ridx + jnp.concatenate([z, ridx[:, : _NB - d]], axis=1)
        d *= 2
    ridx = ridx - 1.0
    rstart_ref[...] = rstart.astype(jnp.int32)[0]
    ridx_ref[...] = jnp.maximum(ridx, 0.0).astype(jnp.int32)[0]

    nz = (padded > 0).astype(jnp.float32)
    rr = nz
    d = 1
    while d < _E:
        z = jnp.zeros((d, 1), jnp.float32)
        rr = rr + jnp.concatenate([z, rr[: _E - d, :]], axis=0)
        d *= 2
    r_e = rr - 1.0
    ii = lax.broadcasted_iota(jnp.int32, (_E, 16), 1).astype(jnp.float32)
    match = jnp.where((r_e == ii) & (padded > 0), 1.0, 0.0)
    eidx = lax.broadcasted_iota(jnp.int32, (_E, 16), 0).astype(jnp.float32)
    raw = jnp.sum(eidx * match, axis=0, keepdims=True)
    has = jnp.sum(match, axis=0, keepdims=True)
    rexp_ref[...] = jnp.where(has > 0, raw, -1.0).astype(jnp.int32)[0]


def _router(x, gate_w):
    return pl.pallas_call(
        _router_body,
        in_specs=[
            pl.BlockSpec((_SEQ, _DIM), lambda: (0, 0)),
            pl.BlockSpec((_E, _DIM), lambda: (0, 0)),
        ],
        out_specs=[
            pl.BlockSpec((_NPAIR,), lambda: (0,)),
            pl.BlockSpec((_NPAIR,), lambda: (0,)),
            pl.BlockSpec((_NB,), lambda: (0,)),
            pl.BlockSpec((_NB,), lambda: (0,)),
            pl.BlockSpec((_NB,), lambda: (0,)),
            pl.BlockSpec((16,), lambda: (0,)),
        ],
        out_shape=[
            jax.ShapeDtypeStruct((_NPAIR,), jnp.int32),
            jax.ShapeDtypeStruct((_NPAIR,), jnp.float32),
            jax.ShapeDtypeStruct((_NB,), jnp.int32),
            jax.ShapeDtypeStruct((_NB,), jnp.int32),
            jax.ShapeDtypeStruct((_NB,), jnp.int32),
            jax.ShapeDtypeStruct((16,), jnp.int32),
        ],
    )(x, gate_w)


_HCH = _JCHUNK // 2


_WCH = _NPAIR // 16
_SSL = _N // _NTILE


def _dispatch_body(x_hbm, pos_hbm, wpair_hbm, xs_hbm, wsort_hbm,
                   idx_v, rows0_v, rows1_v, idxw_v, wpw_v, wsl_v, wsort_sp,
                   s_pos, s_w, s_r0, s_r1, s_x0, s_x1):
    c = lax.axis_index("c")
    s = lax.axis_index("s")
    wid = s * 2 + c
    base = wid * _JCHUNK
    tbase = lax.rem(base, _SEQ)
    wbase = s * _WCH

    cp_r0 = pltpu.async_copy(x_hbm.at[pl.ds(tbase, _HCH), :], rows0_v, s_r0)
    cp_r1 = pltpu.async_copy(x_hbm.at[pl.ds(tbase + _HCH, _HCH), :],
                             rows1_v, s_r1)
    cp_p0 = pltpu.async_copy(pos_hbm.at[pl.ds(base, _HCH)], idx_v.at[0],
                             s_pos)
    cp_p1 = pltpu.async_copy(pos_hbm.at[pl.ds(base + _HCH, _HCH)],
                             idx_v.at[1], s_pos)
    cp_q0 = pltpu.async_copy(pos_hbm.at[pl.ds(wbase, 128)], idxw_v.at[0],
                             s_w)
    cp_q1 = pltpu.async_copy(pos_hbm.at[pl.ds(wbase + 128, 128)],
                             idxw_v.at[1], s_w)
    cp_w0 = pltpu.async_copy(wpair_hbm.at[pl.ds(wbase, 128)], wpw_v.at[0],
                             s_w)
    cp_w1 = pltpu.async_copy(wpair_hbm.at[pl.ds(wbase + 128, 128)],
                             wpw_v.at[1], s_w)
    cp_p0.wait()
    cp_p1.wait()
    cp_r0.wait()
    sc_x0 = pltpu.async_copy(rows0_v, xs_hbm.at[idx_v.at[0]], s_x0)
    cp_r1.wait()
    sc_x1 = pltpu.async_copy(rows1_v, xs_hbm.at[idx_v.at[1]], s_x1)
    cp_q0.wait()
    cp_q1.wait()
    cp_w0.wait()
    cp_w1.wait()
    pltpu.sync_copy(wpw_v.at[0], wsort_sp.at[idxw_v.at[0]])
    pltpu.sync_copy(wpw_v.at[1], wsort_sp.at[idxw_v.at[1]])
    plsc.subcore_barrier()
    pltpu.sync_copy(wsort_sp.at[pl.ds(wid * _SSL, _SSL)], wsl_v)
    pltpu.sync_copy(wsl_v, wsort_hbm.at[pl.ds(wid * _SSL, _SSL)])
    sc_x0.wait()
    sc_x1.wait()


def _dispatch(x, pos, wpair):
    mesh = plsc.VectorSubcoreMesh(core_axis_name="c", subcore_axis_name="s")
    f = pl.kernel(
        _dispatch_body,
        out_type=[
            jax.ShapeDtypeStruct((_N, _DIM), jnp.float32),
            jax.ShapeDtypeStruct((_N,), jnp.float32),
        ],
        mesh=mesh,
        scratch_types=[
            pltpu.VMEM((2, _HCH), jnp.int32),
            pltpu.VMEM((_HCH, _DIM), jnp.float32),
            pltpu.VMEM((_HCH, _DIM), jnp.float32),
            pltpu.VMEM((2, 128), jnp.int32),
            pltpu.VMEM((2, 128), jnp.float32),
            pltpu.VMEM((_SSL,), jnp.float32),
            pltpu.VMEM_SHARED((_N,), jnp.float32),
            pltpu.SemaphoreType.DMA,
            pltpu.SemaphoreType.DMA,
            pltpu.SemaphoreType.DMA,
            pltpu.SemaphoreType.DMA,
            pltpu.SemaphoreType.DMA,
            pltpu.SemaphoreType.DMA,
        ],
    )
    return f(x, pos, wpair)


def _gemm_body(bval_ref, rstart_ref, ridx_ref, rexp_ref,
               xs_ref, ws_ref, w1_hbm, w2_hbm, w3_hbm, yw_ref,
               w1b, w2b, w3b, sems):
    b = pl.program_id(0)
    i = ridx_ref[b]
    slot = lax.rem(i, 2)

    def _issue(run_i, slot_i):
        e = rexp_ref[run_i]

        @pl.when(e >= 0)
        def _():
            pltpu.make_async_copy(w1_hbm.at[e], w1b.at[slot_i],
                                  sems.at[slot_i, 0]).start()
            pltpu.make_async_copy(w2_hbm.at[e], w2b.at[slot_i],
                                  sems.at[slot_i, 1]).start()
            pltpu.make_async_copy(w3_hbm.at[e], w3b.at[slot_i],
                                  sems.at[slot_i, 2]).start()

    def _wait(run_i, slot_i):
        e = jnp.maximum(rexp_ref[run_i], 0)
        pltpu.make_async_copy(w1_hbm.at[e], w1b.at[slot_i],
                              sems.at[slot_i, 0]).wait()
        pltpu.make_async_copy(w2_hbm.at[e], w2b.at[slot_i],
                              sems.at[slot_i, 1]).wait()
        pltpu.make_async_copy(w3_hbm.at[e], w3b.at[slot_i],
                              sems.at[slot_i, 2]).wait()

    @pl.when(b == 0)
    def _():
        _issue(0, 0)
        _issue(1, 1)
        _wait(0, 0)

    @pl.when((rstart_ref[b] == 1) & (b > 0))
    def _():
        _wait(i, slot)
        _issue(i + 1, lax.rem(i + 1, 2))

    @pl.when(bval_ref[b] == 1)
    def _():
        xsb = xs_ref[...]
        h1 = lax.dot_general(xsb, w1b[slot], (((1,), (1,)), ((), ())),
                             preferred_element_type=jnp.float32)
        h3 = lax.dot_general(xsb, w3b[slot], (((1,), (1,)), ((), ())),
                             preferred_element_type=jnp.float32)
        hh = (h1 * (1.0 / (1.0 + jnp.exp(-h1)))) * h3
        y = lax.dot_general(hh, w2b[slot], (((1,), (1,)), ((), ())),
                            preferred_element_type=jnp.float32)
        yw_ref[...] = y * ws_ref[...]


def _gemm(bval, rstart, ridx, rexp, xs, wsort, w1, w2, w3):
    grid_spec = pltpu.PrefetchScalarGridSpec(
        num_scalar_prefetch=4,
        grid=(_NB,),
        in_specs=[
            pl.BlockSpec((_BLK, _DIM), lambda b, *_: (b, 0)),
            pl.BlockSpec((_BLK, 1), lambda b, *_: (b, 0)),
            pl.BlockSpec(memory_space=pl.ANY),
            pl.BlockSpec(memory_space=pl.ANY),
            pl.BlockSpec(memory_space=pl.ANY),
        ],
        out_specs=pl.BlockSpec((_BLK, _DIM), lambda b, *_: (b, 0)),
        scratch_shapes=[
            pltpu.VMEM((2, _HID, _DIM), jnp.float32),
            pltpu.VMEM((2, _DIM, _HID), jnp.float32),
            pltpu.VMEM((2, _HID, _DIM), jnp.float32),
            pltpu.SemaphoreType.DMA((2, 3)),
        ],
    )
    return pl.pallas_call(
        _gemm_body,
        grid_spec=grid_spec,
        out_shape=jax.ShapeDtypeStruct((_N, _DIM), jnp.float32),
    )(bval, rstart, ridx, rexp, xs, wsort, w1, w2, w3)


_TH = _TCHUNK // 2


def _combine_body(yw_hbm, pos_hbm, out_hbm, pa_v, pb_v, za_v, zb_v,
                  s_p, s_a0, s_b0, s_a1, s_b1, s_o):
    c = lax.axis_index("c")
    s = lax.axis_index("s")
    wid = s * 2 + c
    tb = wid * _TCHUNK
    cpa = pltpu.async_copy(pos_hbm.at[pl.ds(tb, _TCHUNK)], pa_v, s_p)
    cpb = pltpu.async_copy(pos_hbm.at[pl.ds(_SEQ + tb, _TCHUNK)], pb_v, s_p)
    cpa.wait()
    cpb.wait()
    g_a0 = pltpu.async_copy(yw_hbm.at[pa_v.at[pl.ds(0, _TH)]],
                            za_v.at[pl.ds(0, _TH), :], s_a0)
    g_b0 = pltpu.async_copy(yw_hbm.at[pb_v.at[pl.ds(0, _TH)]],
                            zb_v.at[pl.ds(0, _TH), :], s_b0)
    g_a1 = pltpu.async_copy(yw_hbm.at[pa_v.at[pl.ds(_TH, _TH)]],
                            za_v.at[pl.ds(_TH, _TH), :], s_a1)
    g_b1 = pltpu.async_copy(yw_hbm.at[pb_v.at[pl.ds(_TH, _TH)]],
                            zb_v.at[pl.ds(_TH, _TH), :], s_b1)

    def row_add(i, carry):
        for d in range(_DIM // 16):
            sl = pl.ds(d * 16, 16)
            za_v[i, sl] = za_v[i, sl] + zb_v[i, sl]
        return carry

    g_a0.wait()
    g_b0.wait()
    lax.fori_loop(0, _TH, row_add, 0)
    w0 = pltpu.async_copy(za_v.at[pl.ds(0, _TH), :],
                          out_hbm.at[pl.ds(tb, _TH), :], s_o)
    g_a1.wait()
    g_b1.wait()
    lax.fori_loop(_TH, _TCHUNK, row_add, 0)
    w0.wait()
    pltpu.sync_copy(za_v.at[pl.ds(_TH, _TH), :],
                    out_hbm.at[pl.ds(tb + _TH, _TH), :])


def _combine(yw, pos):
    mesh = plsc.VectorSubcoreMesh(core_axis_name="c", subcore_axis_name="s")
    f = pl.kernel(
        _combine_body,
        out_type=jax.ShapeDtypeStruct((_SEQ, _DIM), jnp.float32),
        mesh=mesh,
        scratch_types=[
            pltpu.VMEM((_TCHUNK,), jnp.int32),
            pltpu.VMEM((_TCHUNK,), jnp.int32),
            pltpu.VMEM((_TCHUNK, _DIM), jnp.float32),
            pltpu.VMEM((_TCHUNK, _DIM), jnp.float32),
            pltpu.SemaphoreType.DMA,
            pltpu.SemaphoreType.DMA,
            pltpu.SemaphoreType.DMA,
            pltpu.SemaphoreType.DMA,
            pltpu.SemaphoreType.DMA,
            pltpu.SemaphoreType.DMA,
        ],
    )
    return f(yw, pos)


def kernel(hidden_states, gate_w, w1, w2, w3):
    bs, seq, dim = hidden_states.shape
    x = hidden_states.reshape(seq, dim)
    pos, wpair, bval, rstart, ridx, rexp = _router(x, gate_w)
    xs, wsort = _dispatch(x, pos, wpair)
    yw = _gemm(bval, rstart, ridx, rexp, xs, wsort.reshape(_N, 1), w1, w2,
               w3)
    out = _combine(yw, pos)
    return out.reshape(bs, seq, dim)
